# Initial kernel scaffold; baseline (speedup 1.0000x reference)
import jax
import jax.numpy as jnp
from jax.experimental import pallas as pl


def kernel(x, edge_index, etypes, weight, w_comp, loop_weight, h_bias, W1, b1, W2, b2):
    raise NotImplementedError("write your pallas kernel here")


# speedup vs baseline: 12.2364x; 12.2364x over previous
import functools

import jax
import jax.numpy as jnp
from jax import lax
from jax.experimental import pallas as pl
from jax.experimental.pallas import tpu as pltpu
from jax.experimental.pallas import tpu_sc as plsc

N_NODES = 10000
N_EDGES = 320000
FEAT = 128
NUM_RELS = 8
NUM_BASES = 4

NC = 2
NS = 16
NW = NC * NS

K_EDGE = 128
EDGES_PER_TILE = -(-N_EDGES // NW)
CHUNKS_PER_TILE = -(-EDGES_PER_TILE // K_EDGE)
E_PAD = NW * CHUNKS_PER_TILE * K_EDGE

AGG_ROWS = 10112
ROWS_PER_TILE = AGG_ROWS // NS

BN = 400
NB = N_NODES // BN


def _project_kernel(x_ref, wc_ref, w_ref, out_ref):
    r = pl.program_id(1)
    rw = wc_ref[r, 0] * w_ref[0]
    for b in range(1, NUM_BASES):
        rw = rw + wc_ref[r, b] * w_ref[b]
    out_ref[...] = jnp.dot(x_ref[...], rw, preferred_element_type=jnp.float32)


def _mlp_kernel(agg_ref, x_ref, lw_ref, hb_ref, w1_ref, b1_ref, w2_ref, b2_ref,
                out_ref):
    h = agg_ref[0] + agg_ref[1] + hb_ref[...]
    h = h + jnp.dot(x_ref[...], lw_ref[...], preferred_element_type=jnp.float32)
    h = jnp.dot(h, w1_ref[...], preferred_element_type=jnp.float32) + b1_ref[...]
    h = jnp.maximum(h, 0.0)
    h = jnp.dot(h, w2_ref[...], preferred_element_type=jnp.float32) + b2_ref[...]
    out_ref[...] = jnp.maximum(h, 0.0)


def _sc_edge_kernel(ridx_hbm, didx_hbm, hall_hbm, zeros_hbm, out_hbm,
                    ridx_v, didx_v, rows_v, agg_sh, sem):
    c = lax.axis_index("c")
    s = lax.axis_index("s")
    wid = s * NC + c
    row0 = s * ROWS_PER_TILE
    pltpu.sync_copy(zeros_hbm.at[pl.ds(row0, ROWS_PER_TILE)],
                    agg_sh.at[pl.ds(row0, ROWS_PER_TILE)])
    plsc.subcore_barrier()

    def body(j, carry):
        base = (wid * CHUNKS_PER_TILE + j) * K_EDGE
        pltpu.sync_copy(ridx_hbm.at[pl.ds(base, K_EDGE)], ridx_v)
        pltpu.sync_copy(didx_hbm.at[pl.ds(base, K_EDGE)], didx_v)
        pltpu.async_copy(hall_hbm.at[ridx_v], rows_v, sem).wait()
        pltpu.sync_copy(rows_v, agg_sh.at[didx_v], add=True)
        return carry

    lax.fori_loop(0, CHUNKS_PER_TILE, body, 0)
    plsc.subcore_barrier()
    pltpu.sync_copy(agg_sh.at[pl.ds(row0, ROWS_PER_TILE)],
                    out_hbm.at[c, pl.ds(row0, ROWS_PER_TILE)])


_sc_edge = functools.partial(
    pl.kernel,
    out_type=jax.ShapeDtypeStruct((NC, AGG_ROWS, FEAT), jnp.float32),
    mesh=plsc.VectorSubcoreMesh(core_axis_name="c", subcore_axis_name="s"),
    scratch_types=[
        pltpu.VMEM((K_EDGE,), jnp.int32),
        pltpu.VMEM((K_EDGE,), jnp.int32),
        pltpu.VMEM((K_EDGE, FEAT), jnp.float32),
        pltpu.VMEM_SHARED((AGG_ROWS, FEAT), jnp.float32),
        pltpu.SemaphoreType.DMA,
    ],
)(_sc_edge_kernel)


def kernel(x, edge_index, etypes, weight, w_comp, loop_weight, h_bias, W1, b1,
           W2, b2):
    src = edge_index[0].astype(jnp.int32)
    dst = edge_index[1].astype(jnp.int32)
    et = etypes.astype(jnp.int32)

    pad = E_PAD - N_EDGES
    row_idx = jnp.concatenate([et * N_NODES + src,
                               jnp.zeros((pad,), jnp.int32)])
    dst_idx = jnp.concatenate([dst,
                               jnp.full((pad,), N_NODES, jnp.int32)])

    h_all = pl.pallas_call(
        _project_kernel,
        grid=(NB, NUM_RELS),
        in_specs=[
            pl.BlockSpec((BN, FEAT), lambda i, r: (i, 0)),
            pl.BlockSpec(memory_space=pltpu.SMEM),
            pl.BlockSpec((NUM_BASES, FEAT, FEAT), lambda i, r: (0, 0, 0)),
        ],
        out_specs=pl.BlockSpec((BN, FEAT), lambda i, r: (r * NB + i, 0)),
        out_shape=jax.ShapeDtypeStruct((NUM_RELS * N_NODES, FEAT), jnp.float32),
    )(x, w_comp, weight)

    zeros = jnp.zeros((AGG_ROWS, FEAT), jnp.float32)
    agg_parts = _sc_edge(row_idx, dst_idx, h_all, zeros)

    out = pl.pallas_call(
        _mlp_kernel,
        grid=(NB,),
        in_specs=[
            pl.BlockSpec((NC, BN, FEAT), lambda i: (0, i, 0)),
            pl.BlockSpec((BN, FEAT), lambda i: (i, 0)),
            pl.BlockSpec((FEAT, FEAT), lambda i: (0, 0)),
            pl.BlockSpec((1, FEAT), lambda i: (0, 0)),
            pl.BlockSpec((FEAT, FEAT), lambda i: (0, 0)),
            pl.BlockSpec((1, FEAT), lambda i: (0, 0)),
            pl.BlockSpec((FEAT, FEAT), lambda i: (0, 0)),
            pl.BlockSpec((1, FEAT), lambda i: (0, 0)),
        ],
        out_specs=pl.BlockSpec((BN, FEAT), lambda i: (i, 0)),
        out_shape=jax.ShapeDtypeStruct((N_NODES, FEAT), jnp.float32),
    )(agg_parts, x, loop_weight, h_bias.reshape(1, FEAT), W1,
      b1.reshape(1, FEAT), W2, b2.reshape(1, FEAT))
    return out
